# baseline (speedup 1.0000x reference)
import functools

import jax
import jax.numpy as jnp
from jax.experimental import pallas as pl
from jax.experimental.pallas import tpu as pltpu
from jax.experimental.pallas import tpu_sc as plsc

N = 4096
BLK = 512
NB = N // BLK
P = 512
K0 = 400
TAIL = ((400, 300), (300, 200), (200, 100))
F32 = jnp.float32
BF16 = jnp.bfloat16


def _k12_body(h_ref, wt_ref, g_ref, bt_ref, hcol_ref, gbf_ref):
    gblk = g_ref[...]
    gbf = gblk.astype(BF16)
    gbf_ref[...] = gbf
    M = jnp.dot(gbf, h_ref[...].astype(BF16), preferred_element_type=F32)
    s = jnp.dot(M.astype(BF16), wt_ref[...].astype(BF16),
                preferred_element_type=F32)
    hcol_ref[...] = jax.nn.relu(s + bt_ref[0])


def _stage12(g, h, W_top, b_top):
    return pl.pallas_call(
        _k12_body,
        grid=(NB,),
        in_specs=[
            pl.BlockSpec((N, 256), lambda r: (0, 0)),
            pl.BlockSpec((256, 1), lambda r: (0, 0)),
            pl.BlockSpec((BLK, N), lambda r: (r, 0)),
            pl.BlockSpec(memory_space=pltpu.SMEM),
        ],
        out_specs=(pl.BlockSpec((BLK, 1), lambda r: (r, 0)),
                   pl.BlockSpec((BLK, N), lambda r: (r, 0))),
        out_shape=(jax.ShapeDtypeStruct((N, 1), F32),
                   jax.ShapeDtypeStruct((N, N), BF16)),
    )(h, W_top.reshape(256, 1), g, b_top)


def _k3_body(gbf_ref, hs_ref, par_ref, out_ref):
    s = jnp.dot(gbf_ref[...], hs_ref[...], preferred_element_type=F32)
    w = jax.nn.relu(s * par_ref[0] + par_ref[1])
    out_ref[...] = jax.nn.sigmoid(w)


def _stage3(g_bf, h_col, wp0, bp0):
    hs = h_col.astype(BF16)
    par = jnp.concatenate([wp0.reshape(-1), bp0.reshape(-1)])
    return pl.pallas_call(
        _k3_body,
        grid=(NB,),
        in_specs=[
            pl.BlockSpec((BLK, N), lambda r: (r, 0)),
            pl.BlockSpec((N, 1), lambda r: (0, 0)),
            pl.BlockSpec(memory_space=pltpu.SMEM),
        ],
        out_specs=pl.BlockSpec((BLK, 1), lambda r: (r, 0)),
        out_shape=jax.ShapeDtypeStruct((N, 1), F32),
    )(g_bf, hs, par)


def _k4a_body(sblk_ref, sT_ref, rank_ref):
    r = pl.program_id(0)
    s_a = sblk_ref[...]
    s_b = sT_ref[...]
    a_idx = jax.lax.broadcasted_iota(jnp.int32, (BLK, N), 0) + r * BLK
    b_idx = jax.lax.broadcasted_iota(jnp.int32, (BLK, N), 1)
    beats = ((s_b > s_a)
             | ((s_b == s_a) & (b_idx < a_idx))).astype(F32)
    ones = jnp.ones((N, 1), F32)
    rank_ref[...] = jnp.dot(beats, ones, preferred_element_type=F32)


def _stage4a(scores, scoresT):
    return pl.pallas_call(
        _k4a_body,
        grid=(NB,),
        in_specs=[
            pl.BlockSpec((BLK, 1), lambda r: (r, 0)),
            pl.BlockSpec((1, N), lambda r: (0, 0)),
        ],
        out_specs=pl.BlockSpec((BLK, 1), lambda r: (r, 0)),
        out_shape=jax.ShapeDtypeStruct((N, 1), F32),
    )(scores, scoresT)


def _k4b_body(rankT_ref, X_ref, idx_ref, newh_ref):
    rk = rankT_ref[...]
    rr = jax.lax.broadcasted_iota(jnp.int32, (P, N), 0).astype(F32)
    P0 = (rk == rr).astype(F32)
    Y = jnp.dot(P0, X_ref[...], preferred_element_type=F32,
                precision=jax.lax.Precision.HIGHEST)
    idx_f = Y[:, 0:1]
    vals = Y[:, 1:2]
    hsel = Y[:, 2:3]
    rvalid = jax.lax.broadcasted_iota(jnp.int32, (P, 1), 0) < K0
    idx_ref[...] = jnp.where(rvalid, idx_f, 0.0).astype(jnp.int32)
    newh_ref[...] = jnp.where(rvalid, hsel * vals, 0.0)


def _stage4b(rankT, scores, h_col):
    arange = jax.lax.broadcasted_iota(F32, (N, 1), 0)
    X = jnp.concatenate([arange, scores, h_col], axis=1)
    return pl.pallas_call(
        _k4b_body,
        out_shape=(jax.ShapeDtypeStruct((P, 1), jnp.int32),
                   jax.ShapeDtypeStruct((P, 1), F32)),
    )(rankT, X)


def _sc_gather_rows(g, idx):
    info = plsc.get_sparse_core_info()
    nw = info.num_cores * info.num_subcores
    bpw = P // nw
    nw_used = K0 // bpw
    mesh = plsc.VectorSubcoreMesh(core_axis_name="c", subcore_axis_name="s")

    @functools.partial(
        pl.kernel,
        out_type=jax.ShapeDtypeStruct((K0, N), F32),
        mesh=mesh,
        scratch_types=[
            pltpu.VMEM((bpw,), jnp.int32),
            pltpu.VMEM((bpw, N), F32),
            pltpu.SemaphoreType.DMA,
        ],
    )
    def k(table_hbm, idx_hbm, out_hbm, idx_v, rows_v, sem):
        wid = jax.lax.axis_index("s") * info.num_cores + jax.lax.axis_index("c")

        @pl.when(wid < nw_used)
        def _():
            base = wid * bpw
            pltpu.sync_copy(idx_hbm.at[pl.ds(base, bpw)], idx_v)
            pltpu.async_copy(table_hbm.at[idx_v], rows_v, sem).wait()
            pltpu.sync_copy(rows_v, out_hbm.at[pl.ds(base, bpw)])

    return k(g, idx)


def _t_row(colv, ident):
    return jax.lax.dot_general(colv, ident, (((0,), (0,)), ((), ())),
                               preferred_element_type=F32,
                               precision=jax.lax.Precision.HIGHEST)


def _k6_body(R_ref, g_ref, idxT_ref, newh_ref, wd_ref, par_ref,
             o0_ref, o1_ref, o2_ref, o3_ref, facc_ref):
    j = pl.program_id(0)
    Rb = (R_ref[...] != 0).astype(BF16)
    gb = g_ref[...]
    acc = jnp.dot(Rb, gb, preferred_element_type=F32)

    @pl.when(j == 0)
    def _():
        facc_ref[...] = acc

    @pl.when(j > 0)
    def _():
        facc_ref[...] += acc

    @pl.when(j == NB - 1)
    def _():
        row_i = jax.lax.broadcasted_iota(jnp.int32, (P, 1), 0)
        ident = (jax.lax.broadcasted_iota(jnp.int32, (P, P), 0)
                 == jax.lax.broadcasted_iota(jnp.int32, (P, P), 1)
                 ).astype(F32)

        B2 = (facc_ref[...] != 0).astype(BF16)
        jrow = jax.lax.broadcasted_iota(jnp.int32, (N, P), 0)
        bcol = jax.lax.broadcasted_iota(jnp.int32, (N, P), 1)
        S = ((jrow == idxT_ref[...]) & (bcol < K0)).astype(BF16)
        un2 = jnp.dot(B2, S, preferred_element_type=F32)
        un2 = jnp.concatenate(
            [un2, jnp.zeros((P - K0, P), F32)], axis=0)
        ones_p = jnp.ones((P, 1), F32)
        deg = jnp.dot(un2, ones_p, preferred_element_type=F32)
        G = un2 / jnp.maximum(_t_row(deg, ident), 1.0)

        newh = newh_ref[...]
        newh_row = _t_row(newh, ident)
        A = (G.astype(BF16).astype(F32)
             * newh_row.astype(BF16).astype(F32))
        hv = jax.nn.relu(
            jnp.dot(A.astype(BF16), wd_ref[:, 0:1].astype(BF16),
                    preferred_element_type=F32) + par_ref[6])
        hv = hv * (row_i < K0).astype(F32)
        o0_ref[...] = hv[0:400, :]

        outrefs = (None, o1_ref, o2_ref, o3_ref)
        for lvl, (n_prev, kk) in enumerate(TAIL, start=1):
            wp = par_ref[2 * (lvl - 1)]
            bp = par_ref[2 * (lvl - 1) + 1]
            bd = par_ref[6 + lvl]
            s_pre = jnp.dot(G.astype(BF16), hv.astype(BF16),
                            preferred_element_type=F32)
            w = jax.nn.relu(s_pre * wp + bp)
            sc = jnp.where(row_i < n_prev, jax.nn.sigmoid(w), -1.0)
            sc_row = _t_row(sc, ident)
            a_idx = jax.lax.broadcasted_iota(jnp.int32, (P, P), 0)
            b_idx = jax.lax.broadcasted_iota(jnp.int32, (P, P), 1)
            beats = ((sc_row > sc)
                     | ((sc_row == sc) & (b_idx < a_idx))).astype(F32)
            rank = jnp.dot(beats, ones_p,
                           preferred_element_type=F32)
            rank_row = _t_row(rank, ident)
            Psel = ((rank_row == row_i.astype(F32))
                    & (row_i < kk)).astype(F32)
            vals = jnp.dot(Psel, sc, preferred_element_type=F32,
                           precision=jax.lax.Precision.HIGHEST)
            newh_l = jnp.dot(Psel, hv, preferred_element_type=F32,
                             precision=jax.lax.Precision.HIGHEST) * vals
            un = (G != 0).astype(BF16)
            un2f = jnp.dot(un, un, preferred_element_type=F32)
            B2l = (un2f != 0).astype(F32)
            T1 = jnp.dot(Psel, B2l, preferred_element_type=F32)
            un2l = jax.lax.dot_general(T1, Psel, (((1,), (1,)), ((), ())),
                                       preferred_element_type=F32)
            degl = jnp.dot(un2l, ones_p, preferred_element_type=F32)
            G = un2l / jnp.maximum(_t_row(degl, ident), 1.0)
            newh_row = _t_row(newh_l, ident)
            A = (G.astype(BF16).astype(F32)
                 * newh_row.astype(BF16).astype(F32))
            hv = jax.nn.relu(
                jnp.dot(A.astype(BF16), wd_ref[:, lvl:lvl + 1].astype(BF16),
                        preferred_element_type=F32) + bd)
            hv = hv * (row_i < kk).astype(F32)
            outrefs[lvl][...] = hv[0:kk, :]


def _stage6(R, g_bf, idxT, newh, wd_cols, par):
    return pl.pallas_call(
        _k6_body,
        grid=(NB,),
        in_specs=[
            pl.BlockSpec((K0, BLK), lambda j: (0, j)),
            pl.BlockSpec((BLK, N), lambda j: (j, 0)),
            pl.BlockSpec((1, P), lambda j: (0, 0)),
            pl.BlockSpec((P, 1), lambda j: (0, 0)),
            pl.BlockSpec((P, 4), lambda j: (0, 0)),
            pl.BlockSpec(memory_space=pltpu.SMEM),
        ],
        out_specs=(pl.BlockSpec((400, 1), lambda j: (0, 0)),
                   pl.BlockSpec((300, 1), lambda j: (0, 0)),
                   pl.BlockSpec((200, 1), lambda j: (0, 0)),
                   pl.BlockSpec((100, 1), lambda j: (0, 0))),
        out_shape=(jax.ShapeDtypeStruct((400, 1), F32),
                   jax.ShapeDtypeStruct((300, 1), F32),
                   jax.ShapeDtypeStruct((200, 1), F32),
                   jax.ShapeDtypeStruct((100, 1), F32)),
        scratch_shapes=[pltpu.VMEM((K0, N), F32)],
    )(R, g_bf, idxT, newh, wd_cols, par)


def kernel(g, h, W_top, b_top, W_p0, b_p0, W_p1, b_p1, W_p2, b_p2,
           W_p3, b_p3, W_d0, b_d0, W_d1, b_d1, W_d2, b_d2, W_d3, b_d3):
    h_col, g_bf = _stage12(g, h, W_top, b_top)
    scores = _stage3(g_bf, h_col, W_p0, b_p0)
    scoresT = scores.reshape(1, N)
    rank = _stage4a(scores, scoresT)
    idx_col, newh = _stage4b(rank.reshape(1, N), scores, h_col)
    R = _sc_gather_rows(g, idx_col.reshape(P))

    wd_cols = jnp.stack(
        [jnp.pad(W_d0.reshape(-1), (0, P - W_d0.size)),
         jnp.pad(W_d1.reshape(-1), (0, P - W_d1.size)),
         jnp.pad(W_d2.reshape(-1), (0, P - W_d2.size)),
         jnp.pad(W_d3.reshape(-1), (0, P - W_d3.size))], axis=1)
    par = jnp.concatenate([
        W_p1.reshape(-1), b_p1.reshape(-1),
        W_p2.reshape(-1), b_p2.reshape(-1),
        W_p3.reshape(-1), b_p3.reshape(-1),
        b_d0.reshape(-1), b_d1.reshape(-1),
        b_d2.reshape(-1), b_d3.reshape(-1)])

    o0, o1, o2, o3 = _stage6(R, g_bf, idx_col.reshape(1, P), newh,
                             wd_cols, par)
    return jnp.concatenate([h_col, o0, o1, o2, o3], axis=0)


# speedup vs baseline: 1.0223x; 1.0223x over previous
import functools

import jax
import jax.numpy as jnp
from jax.experimental import pallas as pl
from jax.experimental.pallas import tpu as pltpu
from jax.experimental.pallas import tpu_sc as plsc

N = 4096
BLK = 512
NB = N // BLK
P = 512
K0 = 400
TAIL = ((400, 300), (300, 200), (200, 100))
F32 = jnp.float32
BF16 = jnp.bfloat16


def _k12_body(h_ref, wt_ref, g_ref, bt_ref, hcol_ref, gbf_ref):
    gblk = g_ref[...]
    gbf = gblk.astype(BF16)
    gbf_ref[...] = gbf
    M = jnp.dot(gbf, h_ref[...].astype(BF16), preferred_element_type=F32)
    s = jnp.dot(M.astype(BF16), wt_ref[...].astype(BF16),
                preferred_element_type=F32)
    hcol_ref[...] = jax.nn.relu(s + bt_ref[0])


def _stage12(g, h, W_top, b_top):
    return pl.pallas_call(
        _k12_body,
        grid=(NB,),
        in_specs=[
            pl.BlockSpec((N, 256), lambda r: (0, 0)),
            pl.BlockSpec((256, 1), lambda r: (0, 0)),
            pl.BlockSpec((BLK, N), lambda r: (r, 0)),
            pl.BlockSpec(memory_space=pltpu.SMEM),
        ],
        out_specs=(pl.BlockSpec((BLK, 1), lambda r: (r, 0)),
                   pl.BlockSpec((BLK, N), lambda r: (r, 0))),
        out_shape=(jax.ShapeDtypeStruct((N, 1), F32),
                   jax.ShapeDtypeStruct((N, N), BF16)),
    )(h, W_top.reshape(256, 1), g, b_top)


def _k3_body(gbf_ref, hs_ref, par_ref, out_ref):
    s = jnp.dot(gbf_ref[...], hs_ref[...], preferred_element_type=F32)
    w = jax.nn.relu(s * par_ref[0] + par_ref[1])
    out_ref[...] = jax.nn.sigmoid(w)


def _stage3(g_bf, h_col, wp0, bp0):
    hs = h_col.astype(BF16)
    par = jnp.concatenate([wp0.reshape(-1), bp0.reshape(-1)])
    return pl.pallas_call(
        _k3_body,
        grid=(NB,),
        in_specs=[
            pl.BlockSpec((BLK, N), lambda r: (r, 0)),
            pl.BlockSpec((N, 1), lambda r: (0, 0)),
            pl.BlockSpec(memory_space=pltpu.SMEM),
        ],
        out_specs=pl.BlockSpec((BLK, 1), lambda r: (r, 0)),
        out_shape=jax.ShapeDtypeStruct((N, 1), F32),
    )(g_bf, hs, par)


def _k4a_body(sblk_ref, sT_ref, rank_ref):
    r = pl.program_id(0)
    s_a = sblk_ref[...]
    s_b = sT_ref[...]
    a_idx = jax.lax.broadcasted_iota(jnp.int32, (BLK, N), 0) + r * BLK
    b_idx = jax.lax.broadcasted_iota(jnp.int32, (BLK, N), 1)
    gt = (s_b > s_a).astype(F32)
    eq = ((s_b == s_a) & (b_idx < a_idx)).astype(F32)
    ones = jnp.ones((N, 1), F32)
    rank_ref[...] = jnp.dot(gt + eq, ones, preferred_element_type=F32)


def _stage4a(scores, scoresT):
    return pl.pallas_call(
        _k4a_body,
        grid=(NB,),
        in_specs=[
            pl.BlockSpec((BLK, 1), lambda r: (r, 0)),
            pl.BlockSpec((1, N), lambda r: (0, 0)),
        ],
        out_specs=pl.BlockSpec((BLK, 1), lambda r: (r, 0)),
        out_shape=jax.ShapeDtypeStruct((N, 1), F32),
    )(scores, scoresT)


def _k4b_body(rankT_ref, X_ref, idx_ref, newh_ref):
    rk = rankT_ref[...]
    rr = jax.lax.broadcasted_iota(jnp.int32, (P, N), 0).astype(F32)
    P0 = (rk == rr).astype(F32)
    Y = jnp.dot(P0, X_ref[...], preferred_element_type=F32,
                precision=jax.lax.Precision.HIGHEST)
    idx_f = Y[:, 0:1]
    vals = Y[:, 1:2]
    hsel = Y[:, 2:3]
    rvalid = jax.lax.broadcasted_iota(jnp.int32, (P, 1), 0) < K0
    idx_ref[...] = jnp.where(rvalid, idx_f, 0.0).astype(jnp.int32)
    newh_ref[...] = jnp.where(rvalid, hsel * vals, 0.0)


def _stage4b(rankT, scores, h_col):
    arange = jax.lax.broadcasted_iota(F32, (N, 1), 0)
    X = jnp.concatenate([arange, scores, h_col], axis=1)
    return pl.pallas_call(
        _k4b_body,
        out_shape=(jax.ShapeDtypeStruct((P, 1), jnp.int32),
                   jax.ShapeDtypeStruct((P, 1), F32)),
    )(rankT, X)


def _sc_gather_rows(g, idx):
    info = plsc.get_sparse_core_info()
    nw = info.num_cores * info.num_subcores
    bpw = P // nw
    nw_used = K0 // bpw
    mesh = plsc.VectorSubcoreMesh(core_axis_name="c", subcore_axis_name="s")

    @functools.partial(
        pl.kernel,
        out_type=jax.ShapeDtypeStruct((K0, N), F32),
        mesh=mesh,
        scratch_types=[
            pltpu.VMEM((bpw,), jnp.int32),
            pltpu.VMEM((bpw, N), F32),
            pltpu.SemaphoreType.DMA,
        ],
    )
    def k(table_hbm, idx_hbm, out_hbm, idx_v, rows_v, sem):
        wid = jax.lax.axis_index("s") * info.num_cores + jax.lax.axis_index("c")

        @pl.when(wid < nw_used)
        def _():
            base = wid * bpw
            pltpu.sync_copy(idx_hbm.at[pl.ds(base, bpw)], idx_v)
            pltpu.async_copy(table_hbm.at[idx_v], rows_v, sem).wait()
            pltpu.sync_copy(rows_v, out_hbm.at[pl.ds(base, bpw)])

    return k(g, idx)


def _t_row(colv, ident):
    return jax.lax.dot_general(colv, ident, (((0,), (0,)), ((), ())),
                               preferred_element_type=F32,
                               precision=jax.lax.Precision.HIGHEST)


def _k6_body(R_ref, g_ref, idxT_ref, newh_ref, wd_ref, par_ref,
             o0_ref, o1_ref, o2_ref, o3_ref, facc_ref):
    j = pl.program_id(0)
    Rb = (R_ref[...] != 0).astype(BF16)
    gb = g_ref[...]
    acc = jnp.dot(Rb, gb, preferred_element_type=F32)

    @pl.when(j == 0)
    def _():
        facc_ref[...] = acc

    @pl.when(j > 0)
    def _():
        facc_ref[...] += acc

    @pl.when(j == NB - 1)
    def _():
        row_i = jax.lax.broadcasted_iota(jnp.int32, (P, 1), 0)
        ident = (jax.lax.broadcasted_iota(jnp.int32, (P, P), 0)
                 == jax.lax.broadcasted_iota(jnp.int32, (P, P), 1)
                 ).astype(F32)

        B2 = (facc_ref[...] != 0).astype(BF16)
        jrow = jax.lax.broadcasted_iota(jnp.int32, (N, P), 0)
        bcol = jax.lax.broadcasted_iota(jnp.int32, (N, P), 1)
        S = ((jrow == idxT_ref[...]) & (bcol < K0)).astype(BF16)
        un2 = jnp.dot(B2, S, preferred_element_type=F32)
        un2 = jnp.concatenate(
            [un2, jnp.zeros((P - K0, P), F32)], axis=0)
        ones_p = jnp.ones((P, 1), F32)
        deg = jnp.dot(un2, ones_p, preferred_element_type=F32)
        G = un2 / jnp.maximum(_t_row(deg, ident), 1.0)

        newh = newh_ref[...]
        newh_row = _t_row(newh, ident)
        A = (G.astype(BF16).astype(F32)
             * newh_row.astype(BF16).astype(F32))
        hv = jax.nn.relu(
            jnp.dot(A.astype(BF16), wd_ref[:, 0:1].astype(BF16),
                    preferred_element_type=F32) + par_ref[6])
        hv = hv * (row_i < K0).astype(F32)
        o0_ref[...] = hv[0:400, :]

        outrefs = (None, o1_ref, o2_ref, o3_ref)
        for lvl, (n_prev, kk) in enumerate(TAIL, start=1):
            wp = par_ref[2 * (lvl - 1)]
            bp = par_ref[2 * (lvl - 1) + 1]
            bd = par_ref[6 + lvl]
            s_pre = jnp.dot(G.astype(BF16), hv.astype(BF16),
                            preferred_element_type=F32)
            w = jax.nn.relu(s_pre * wp + bp)
            sc = jnp.where(row_i < n_prev, jax.nn.sigmoid(w), -1.0)
            sc_row = _t_row(sc, ident)
            a_idx = jax.lax.broadcasted_iota(jnp.int32, (P, P), 0)
            b_idx = jax.lax.broadcasted_iota(jnp.int32, (P, P), 1)
            gt = (sc_row > sc).astype(F32)
            eq = ((sc_row == sc) & (b_idx < a_idx)).astype(F32)
            rank = jnp.dot(gt + eq, ones_p,
                           preferred_element_type=F32)
            rank_row = _t_row(rank, ident)
            Psel = ((rank_row == row_i.astype(F32))
                    & (row_i < kk)).astype(F32)
            vals = jnp.dot(Psel, sc, preferred_element_type=F32,
                           precision=jax.lax.Precision.HIGHEST)
            newh_l = jnp.dot(Psel, hv, preferred_element_type=F32,
                             precision=jax.lax.Precision.HIGHEST) * vals
            un = (G != 0).astype(BF16)
            un2f = jnp.dot(un, un, preferred_element_type=F32)
            B2l = (un2f != 0).astype(F32)
            T1 = jnp.dot(Psel, B2l, preferred_element_type=F32)
            un2l = jax.lax.dot_general(T1, Psel, (((1,), (1,)), ((), ())),
                                       preferred_element_type=F32)
            degl = jnp.dot(un2l, ones_p, preferred_element_type=F32)
            G = un2l / jnp.maximum(_t_row(degl, ident), 1.0)
            newh_row = _t_row(newh_l, ident)
            A = (G.astype(BF16).astype(F32)
                 * newh_row.astype(BF16).astype(F32))
            hv = jax.nn.relu(
                jnp.dot(A.astype(BF16), wd_ref[:, lvl:lvl + 1].astype(BF16),
                        preferred_element_type=F32) + bd)
            hv = hv * (row_i < kk).astype(F32)
            outrefs[lvl][...] = hv[0:kk, :]


def _stage6(R, g_bf, idxT, newh, wd_cols, par):
    return pl.pallas_call(
        _k6_body,
        grid=(NB,),
        in_specs=[
            pl.BlockSpec((K0, BLK), lambda j: (0, j)),
            pl.BlockSpec((BLK, N), lambda j: (j, 0)),
            pl.BlockSpec((1, P), lambda j: (0, 0)),
            pl.BlockSpec((P, 1), lambda j: (0, 0)),
            pl.BlockSpec((P, 4), lambda j: (0, 0)),
            pl.BlockSpec(memory_space=pltpu.SMEM),
        ],
        out_specs=(pl.BlockSpec((400, 1), lambda j: (0, 0)),
                   pl.BlockSpec((300, 1), lambda j: (0, 0)),
                   pl.BlockSpec((200, 1), lambda j: (0, 0)),
                   pl.BlockSpec((100, 1), lambda j: (0, 0))),
        out_shape=(jax.ShapeDtypeStruct((400, 1), F32),
                   jax.ShapeDtypeStruct((300, 1), F32),
                   jax.ShapeDtypeStruct((200, 1), F32),
                   jax.ShapeDtypeStruct((100, 1), F32)),
        scratch_shapes=[pltpu.VMEM((K0, N), F32)],
    )(R, g_bf, idxT, newh, wd_cols, par)


def kernel(g, h, W_top, b_top, W_p0, b_p0, W_p1, b_p1, W_p2, b_p2,
           W_p3, b_p3, W_d0, b_d0, W_d1, b_d1, W_d2, b_d2, W_d3, b_d3):
    h_col, g_bf = _stage12(g, h, W_top, b_top)
    scores = _stage3(g_bf, h_col, W_p0, b_p0)
    scoresT = scores.reshape(1, N)
    rank = _stage4a(scores, scoresT)
    idx_col, newh = _stage4b(rank.reshape(1, N), scores, h_col)
    R = _sc_gather_rows(g, idx_col.reshape(P))

    wd_cols = jnp.stack(
        [jnp.pad(W_d0.reshape(-1), (0, P - W_d0.size)),
         jnp.pad(W_d1.reshape(-1), (0, P - W_d1.size)),
         jnp.pad(W_d2.reshape(-1), (0, P - W_d2.size)),
         jnp.pad(W_d3.reshape(-1), (0, P - W_d3.size))], axis=1)
    par = jnp.concatenate([
        W_p1.reshape(-1), b_p1.reshape(-1),
        W_p2.reshape(-1), b_p2.reshape(-1),
        W_p3.reshape(-1), b_p3.reshape(-1),
        b_d0.reshape(-1), b_d1.reshape(-1),
        b_d2.reshape(-1), b_d3.reshape(-1)])

    o0, o1, o2, o3 = _stage6(R, g_bf, idx_col.reshape(1, P), newh,
                             wd_cols, par)
    return jnp.concatenate([h_col, o0, o1, o2, o3], axis=0)


# speedup vs baseline: 1.0383x; 1.0157x over previous
import functools

import jax
import jax.numpy as jnp
from jax.experimental import pallas as pl
from jax.experimental.pallas import tpu as pltpu
from jax.experimental.pallas import tpu_sc as plsc

N = 4096
BLK = 512
NB = N // BLK
P = 512
K0 = 400
TAIL = ((400, 300), (300, 200), (200, 100))
F32 = jnp.float32
BF16 = jnp.bfloat16


def _k12_body(h_ref, wt_ref, g_ref, bt_ref, hcol_ref, gbf_ref):
    gblk = g_ref[...]
    gbf = gblk.astype(BF16)
    gbf_ref[...] = gbf
    M = jnp.dot(gbf, h_ref[...].astype(BF16), preferred_element_type=F32)
    s = jnp.dot(M.astype(BF16), wt_ref[...].astype(BF16),
                preferred_element_type=F32)
    hcol_ref[...] = jax.nn.relu(s + bt_ref[0])


def _stage12(g, h, W_top, b_top):
    return pl.pallas_call(
        _k12_body,
        grid=(NB,),
        in_specs=[
            pl.BlockSpec((N, 256), lambda r: (0, 0)),
            pl.BlockSpec((256, 1), lambda r: (0, 0)),
            pl.BlockSpec((BLK, N), lambda r: (r, 0)),
            pl.BlockSpec(memory_space=pltpu.SMEM),
        ],
        out_specs=(pl.BlockSpec((BLK, 1), lambda r: (r, 0)),
                   pl.BlockSpec((BLK, N), lambda r: (r, 0))),
        out_shape=(jax.ShapeDtypeStruct((N, 1), F32),
                   jax.ShapeDtypeStruct((N, N), BF16)),
    )(h, W_top.reshape(256, 1), g, b_top)


def _t_col(rowv, ident):
    return jax.lax.dot_general(ident, rowv, (((1,), (1,)), ((), ())),
                               preferred_element_type=F32,
                               precision=jax.lax.Precision.HIGHEST)


def _k345_body(gbf_ref, hs_ref, hcol_ref, par_ref, idx_ref, newh_ref,
               sT8, rT8):
    t = pl.program_id(0)
    ident = (jax.lax.broadcasted_iota(jnp.int32, (BLK, BLK), 0)
             == jax.lax.broadcasted_iota(jnp.int32, (BLK, BLK), 1)
             ).astype(F32)
    ones = jnp.ones((BLK, 1), F32)

    @pl.when(t < NB)
    def _():
        s = jnp.dot(gbf_ref[...], hs_ref[...], preferred_element_type=F32)
        w = jax.nn.relu(s * par_ref[0] + par_ref[1])
        sT8[pl.ds(t, 1), :] = _t_row(jax.nn.sigmoid(w), ident)

    @pl.when((t >= NB) & (t < 2 * NB))
    def _():
        r = t - NB
        s_a = _t_col(sT8[pl.ds(r, 1), :], ident)
        acc = jnp.zeros((BLK, 1), F32)
        for cc in range(NB):
            s_b = sT8[cc:cc + 1, :]
            a_idx = (jax.lax.broadcasted_iota(jnp.int32, (BLK, BLK), 0)
                     + r * BLK)
            b_idx = (jax.lax.broadcasted_iota(jnp.int32, (BLK, BLK), 1)
                     + cc * BLK)
            gt = (s_b > s_a).astype(F32)
            eq = ((s_b == s_a) & (b_idx < a_idx)).astype(F32)
            acc += jnp.dot(gt + eq, ones, preferred_element_type=F32)
        rT8[pl.ds(r, 1), :] = _t_row(acc, ident)

    @pl.when(t == 2 * NB)
    def _():
        slot = jax.lax.broadcasted_iota(jnp.int32, (BLK, 1), 0)
        slot_f = slot.astype(F32)
        Y = jnp.zeros((BLK, 3), F32)
        for cc in range(NB):
            rank_row = rT8[cc:cc + 1, :]
            P0 = (rank_row == slot_f).astype(F32)
            sc_col = _t_col(sT8[cc:cc + 1, :], ident)
            ar = (jax.lax.broadcasted_iota(jnp.int32, (BLK, 1), 0)
                  + cc * BLK).astype(F32)
            hcc = hcol_ref[cc * BLK:(cc + 1) * BLK, :]
            Xcc = jnp.concatenate([ar, sc_col, hcc], axis=1)
            Y += jnp.dot(P0, Xcc, preferred_element_type=F32,
                         precision=jax.lax.Precision.HIGHEST)
        idx_f = Y[:, 0:1]
        vals = Y[:, 1:2]
        hsel = Y[:, 2:3]
        rvalid = slot < K0
        idx_ref[...] = jnp.where(rvalid, idx_f, 0.0).astype(jnp.int32)
        newh_ref[...] = jnp.where(rvalid, hsel * vals, 0.0)


def _stage345(g_bf, h_col, wp0, bp0):
    hs = h_col.astype(BF16)
    par = jnp.concatenate([wp0.reshape(-1), bp0.reshape(-1)])
    return pl.pallas_call(
        _k345_body,
        grid=(2 * NB + 1,),
        in_specs=[
            pl.BlockSpec((BLK, N), lambda t: (jnp.minimum(t, NB - 1), 0)),
            pl.BlockSpec((N, 1), lambda t: (0, 0)),
            pl.BlockSpec((N, 1), lambda t: (0, 0)),
            pl.BlockSpec(memory_space=pltpu.SMEM),
        ],
        out_specs=(pl.BlockSpec((P, 1), lambda t: (0, 0)),
                   pl.BlockSpec((P, 1), lambda t: (0, 0))),
        out_shape=(jax.ShapeDtypeStruct((P, 1), jnp.int32),
                   jax.ShapeDtypeStruct((P, 1), F32)),
        scratch_shapes=[pltpu.VMEM((NB, BLK), F32),
                        pltpu.VMEM((NB, BLK), F32)],
    )(g_bf, hs, h_col, par)


def _sc_gather_rows(g, idx):
    info = plsc.get_sparse_core_info()
    nw = info.num_cores * info.num_subcores
    bpw = P // nw
    nw_used = K0 // bpw
    mesh = plsc.VectorSubcoreMesh(core_axis_name="c", subcore_axis_name="s")

    @functools.partial(
        pl.kernel,
        out_type=jax.ShapeDtypeStruct((K0, N), F32),
        mesh=mesh,
        scratch_types=[
            pltpu.VMEM((bpw,), jnp.int32),
            pltpu.VMEM((bpw, N), F32),
            pltpu.SemaphoreType.DMA,
        ],
    )
    def k(table_hbm, idx_hbm, out_hbm, idx_v, rows_v, sem):
        wid = jax.lax.axis_index("s") * info.num_cores + jax.lax.axis_index("c")

        @pl.when(wid < nw_used)
        def _():
            base = wid * bpw
            pltpu.sync_copy(idx_hbm.at[pl.ds(base, bpw)], idx_v)
            pltpu.async_copy(table_hbm.at[idx_v], rows_v, sem).wait()
            pltpu.sync_copy(rows_v, out_hbm.at[pl.ds(base, bpw)])

    return k(g, idx)


def _t_row(colv, ident):
    return jax.lax.dot_general(colv, ident, (((0,), (0,)), ((), ())),
                               preferred_element_type=F32,
                               precision=jax.lax.Precision.HIGHEST)


def _k6_body(R_ref, g_ref, idxT_ref, newh_ref, wd_ref, par_ref,
             o0_ref, o1_ref, o2_ref, o3_ref, facc_ref):
    j = pl.program_id(0)
    Rb = (R_ref[...] != 0).astype(BF16)
    gb = g_ref[...]
    acc = jnp.dot(Rb, gb, preferred_element_type=F32)

    @pl.when(j == 0)
    def _():
        facc_ref[...] = acc

    @pl.when(j > 0)
    def _():
        facc_ref[...] += acc

    @pl.when(j == NB - 1)
    def _():
        row_i = jax.lax.broadcasted_iota(jnp.int32, (P, 1), 0)
        ident = (jax.lax.broadcasted_iota(jnp.int32, (P, P), 0)
                 == jax.lax.broadcasted_iota(jnp.int32, (P, P), 1)
                 ).astype(F32)

        B2 = (facc_ref[...] != 0).astype(BF16)
        jrow = jax.lax.broadcasted_iota(jnp.int32, (N, P), 0)
        bcol = jax.lax.broadcasted_iota(jnp.int32, (N, P), 1)
        S = ((jrow == idxT_ref[...]) & (bcol < K0)).astype(BF16)
        un2 = jnp.dot(B2, S, preferred_element_type=F32)
        un2 = jnp.concatenate(
            [un2, jnp.zeros((P - K0, P), F32)], axis=0)
        ones_p = jnp.ones((P, 1), F32)
        deg = jnp.dot(un2, ones_p, preferred_element_type=F32)
        G = un2 / jnp.maximum(_t_row(deg, ident), 1.0)

        newh = newh_ref[...]
        newh_row = _t_row(newh, ident)
        A = (G.astype(BF16).astype(F32)
             * newh_row.astype(BF16).astype(F32))
        hv = jax.nn.relu(
            jnp.dot(A.astype(BF16), wd_ref[:, 0:1].astype(BF16),
                    preferred_element_type=F32) + par_ref[6])
        hv = hv * (row_i < K0).astype(F32)
        o0_ref[...] = hv[0:400, :]

        outrefs = (None, o1_ref, o2_ref, o3_ref)
        for lvl, (n_prev, kk) in enumerate(TAIL, start=1):
            wp = par_ref[2 * (lvl - 1)]
            bp = par_ref[2 * (lvl - 1) + 1]
            bd = par_ref[6 + lvl]
            s_pre = jnp.dot(G.astype(BF16), hv.astype(BF16),
                            preferred_element_type=F32)
            w = jax.nn.relu(s_pre * wp + bp)
            sc = jnp.where(row_i < n_prev, jax.nn.sigmoid(w), -1.0)
            sc_row = _t_row(sc, ident)
            a_idx = jax.lax.broadcasted_iota(jnp.int32, (P, P), 0)
            b_idx = jax.lax.broadcasted_iota(jnp.int32, (P, P), 1)
            gt = (sc_row > sc).astype(F32)
            eq = ((sc_row == sc) & (b_idx < a_idx)).astype(F32)
            rank = jnp.dot(gt + eq, ones_p,
                           preferred_element_type=F32)
            rank_row = _t_row(rank, ident)
            Psel = ((rank_row == row_i.astype(F32))
                    & (row_i < kk)).astype(F32)
            vals = jnp.dot(Psel, sc, preferred_element_type=F32,
                           precision=jax.lax.Precision.HIGHEST)
            newh_l = jnp.dot(Psel, hv, preferred_element_type=F32,
                             precision=jax.lax.Precision.HIGHEST) * vals
            un = (G != 0).astype(BF16)
            un2f = jnp.dot(un, un, preferred_element_type=F32)
            B2l = (un2f != 0).astype(F32)
            T1 = jnp.dot(Psel, B2l, preferred_element_type=F32)
            un2l = jax.lax.dot_general(T1, Psel, (((1,), (1,)), ((), ())),
                                       preferred_element_type=F32)
            degl = jnp.dot(un2l, ones_p, preferred_element_type=F32)
            G = un2l / jnp.maximum(_t_row(degl, ident), 1.0)
            newh_row = _t_row(newh_l, ident)
            A = (G.astype(BF16).astype(F32)
                 * newh_row.astype(BF16).astype(F32))
            hv = jax.nn.relu(
                jnp.dot(A.astype(BF16), wd_ref[:, lvl:lvl + 1].astype(BF16),
                        preferred_element_type=F32) + bd)
            hv = hv * (row_i < kk).astype(F32)
            outrefs[lvl][...] = hv[0:kk, :]


def _stage6(R, g_bf, idxT, newh, wd_cols, par):
    return pl.pallas_call(
        _k6_body,
        grid=(NB,),
        in_specs=[
            pl.BlockSpec((K0, BLK), lambda j: (0, j)),
            pl.BlockSpec((BLK, N), lambda j: (j, 0)),
            pl.BlockSpec((1, P), lambda j: (0, 0)),
            pl.BlockSpec((P, 1), lambda j: (0, 0)),
            pl.BlockSpec((P, 4), lambda j: (0, 0)),
            pl.BlockSpec(memory_space=pltpu.SMEM),
        ],
        out_specs=(pl.BlockSpec((400, 1), lambda j: (0, 0)),
                   pl.BlockSpec((300, 1), lambda j: (0, 0)),
                   pl.BlockSpec((200, 1), lambda j: (0, 0)),
                   pl.BlockSpec((100, 1), lambda j: (0, 0))),
        out_shape=(jax.ShapeDtypeStruct((400, 1), F32),
                   jax.ShapeDtypeStruct((300, 1), F32),
                   jax.ShapeDtypeStruct((200, 1), F32),
                   jax.ShapeDtypeStruct((100, 1), F32)),
        scratch_shapes=[pltpu.VMEM((K0, N), F32)],
    )(R, g_bf, idxT, newh, wd_cols, par)


def kernel(g, h, W_top, b_top, W_p0, b_p0, W_p1, b_p1, W_p2, b_p2,
           W_p3, b_p3, W_d0, b_d0, W_d1, b_d1, W_d2, b_d2, W_d3, b_d3):
    h_col, g_bf = _stage12(g, h, W_top, b_top)
    idx_col, newh = _stage345(g_bf, h_col, W_p0, b_p0)
    R = _sc_gather_rows(g, idx_col.reshape(P))

    wd_cols = jnp.stack(
        [jnp.pad(W_d0.reshape(-1), (0, P - W_d0.size)),
         jnp.pad(W_d1.reshape(-1), (0, P - W_d1.size)),
         jnp.pad(W_d2.reshape(-1), (0, P - W_d2.size)),
         jnp.pad(W_d3.reshape(-1), (0, P - W_d3.size))], axis=1)
    par = jnp.concatenate([
        W_p1.reshape(-1), b_p1.reshape(-1),
        W_p2.reshape(-1), b_p2.reshape(-1),
        W_p3.reshape(-1), b_p3.reshape(-1),
        b_d0.reshape(-1), b_d1.reshape(-1),
        b_d2.reshape(-1), b_d3.reshape(-1)])

    o0, o1, o2, o3 = _stage6(R, g_bf, idx_col.reshape(1, P), newh,
                             wd_cols, par)
    return jnp.concatenate([h_col, o0, o1, o2, o3], axis=0)
